# baseline (speedup 1.0000x reference)
import functools

import jax
import jax.numpy as jnp
from jax.experimental import pallas as pl
from jax.experimental.pallas import tpu as pltpu

N = 50; T = 100; B = 16; C = 32; H = 512; DH = 150; OUT = 50
NP = 64
NC = N * C
G3 = 3 * H
DP = 256
G3D = 3 * DP
OUTP = 128
BT = B * T
CHUNK = 160
NCHUNK = BT // CHUNK
TCH = CHUNK // B


def _prep_kernel(emb_ref, ew_ref, ev_ref, g1_ref):
    emb = emb_ref[...]
    sq = jnp.sum(emb * emb, axis=1, keepdims=True)
    norm = jnp.maximum(jnp.sqrt(sq), 1e-8)
    wn = emb / norm
    a = jax.lax.dot_general(wn, wn, (((1,), (1,)), ((), ())),
                            preferred_element_type=jnp.float32)
    ii = jax.lax.broadcasted_iota(jnp.int32, (NP, NP), 0)
    jj = jax.lax.broadcasted_iota(jnp.int32, (NP, NP), 1)
    a = jnp.where(ii == jj, 0.0, a)
    a = jnp.maximum(a, 0.0)
    deg = jnp.sum(a, axis=0, keepdims=True)
    dis = jnp.where(deg > 0, jax.lax.rsqrt(deg), 0.0)
    m = a * dis * jnp.transpose(dis)
    g1_ref[...] = jnp.dot(m, ew_ref[...],
                          preferred_element_type=jnp.float32) + ev_ref[...]


def _gemmenc_kernel(x_ref, g1_ref, brow_ref, wih_ref, bih_ref, whh_ref,
                    bhh_ref, hend_ref, gibuf_ref, h_ref):
    k = pl.program_id(0)

    @pl.when(k == 0)
    def _():
        h_ref[...] = jnp.zeros((B, H), jnp.float32)

    y = jnp.dot(x_ref[...], g1_ref[...], preferred_element_type=jnp.float32)
    y = y + brow_ref[...]
    g = 0.5 * y * (1.0 + jax.lax.erf(y * 0.7071067811865476))
    gibuf_ref[k % 2] = (jax.lax.dot_general(
        g, wih_ref[...], (((1,), (1,)), ((), ())),
        preferred_element_type=jnp.float32) + bih_ref[...]).reshape(TCH, B, G3)

    def step(i, h):
        gi = gibuf_ref[(k - 1) % 2, i]
        gh = jax.lax.dot_general(
            h.astype(jnp.bfloat16), whh_ref[...], (((1,), (1,)), ((), ())),
            preferred_element_type=jnp.float32) + bhh_ref[...]
        r = jax.nn.sigmoid(gi[:, :H] + gh[:, :H])
        z = jax.nn.sigmoid(gi[:, H:2 * H] + gh[:, H:2 * H])
        n = jnp.tanh(gi[:, 2 * H:] + r * gh[:, 2 * H:])
        return (1.0 - z) * n + z * h

    h = jax.lax.fori_loop(0, TCH, step, h_ref[...])
    h_ref[...] = jnp.where(k == 0, jnp.zeros((B, H), jnp.float32), h)

    @pl.when(k == NCHUNK)
    def _():
        hend_ref[...] = h_ref[...]


def _dec_kernel(rep_ref, dwih_ref, dbih_ref, dwhh_ref, dbhh_ref, fcw_ref,
                fcb_ref, out_ref, gid_ref, hs_ref):
    gid_ref[...] = (jnp.dot(rep_ref[...], dwih_ref[...],
                            preferred_element_type=jnp.float32)
                    + dbih_ref[...]).reshape(T, B, G3D)

    def dstep(t, hd):
        gi = gid_ref[t]
        ghd = jnp.dot(hd.astype(jnp.bfloat16), dwhh_ref[...],
                      preferred_element_type=jnp.float32) + dbhh_ref[...]
        r = jax.nn.sigmoid(gi[:, :DP] + ghd[:, :DP])
        z = jax.nn.sigmoid(gi[:, DP:2 * DP] + ghd[:, DP:2 * DP])
        n = jnp.tanh(gi[:, 2 * DP:] + r * ghd[:, 2 * DP:])
        hd = (1.0 - z) * n + z * hd
        hs_ref[t] = hd
        return hd

    jax.lax.fori_loop(0, T, dstep, jnp.zeros((B, DP), jnp.float32))
    out_ref[...] = jnp.dot(hs_ref[...].reshape(BT, DP), fcw_ref[...],
                           preferred_element_type=jnp.float32) + fcb_ref[...]


def _pad2(x, r, c):
    return jnp.pad(x, ((0, r - x.shape[0]), (0, c - x.shape[1])))


@jax.jit
def kernel(window, emb_W, arma_w, arma_v, arma_b, gru_Wih, gru_Whh, gru_bih,
           gru_bhh, dec_Wih, dec_Whh, dec_bih, dec_bhh, fc_W, fc_b):
    f32 = jnp.float32
    xtb = jnp.transpose(window, (1, 0, 2)).reshape(BT, N)
    x_pad = _pad2(xtb, BT, NP)
    emb_pad = _pad2(emb_W, NP, 128)
    eye = jnp.eye(N, dtype=f32)
    ew = _pad2((eye[:, :, None] * arma_w[0][None, None, :]).reshape(N, NC), NP, NC)
    ev = _pad2((eye[:, :, None] * arma_v[0][None, None, :]).reshape(N, NC), NP, NC)
    brow = jnp.tile(arma_b, N)[None, :]
    bih = gru_bih[None, :]
    whh_bf = gru_Whh.astype(jnp.bfloat16)
    bhh = gru_bhh[None, :]
    dwihT = jnp.concatenate(
        [_pad2(dec_Wih[g * DH:(g + 1) * DH, :].T, H, DP) for g in range(3)],
        axis=1)
    dbih = jnp.concatenate(
        [jnp.pad(dec_bih[g * DH:(g + 1) * DH], (0, DP - DH)) for g in range(3)]
    )[None, :]
    dwhhT = jnp.concatenate(
        [_pad2(dec_Whh[g * DH:(g + 1) * DH, :].T, DP, DP) for g in range(3)],
        axis=1).astype(jnp.bfloat16)
    dbhh = jnp.concatenate(
        [jnp.pad(dec_bhh[g * DH:(g + 1) * DH], (0, DP - DH)) for g in range(3)]
    )[None, :]
    fcwT = _pad2(fc_W.T, DP, OUTP)
    fcb = jnp.pad(fc_b, (0, OUTP - OUT))[None, :]

    g1 = pl.pallas_call(
        _prep_kernel,
        out_shape=jax.ShapeDtypeStruct((NP, NC), f32),
    )(emb_pad, ew, ev)

    h_end = pl.pallas_call(
        _gemmenc_kernel,
        grid=(NCHUNK + 1,),
        in_specs=[
            pl.BlockSpec((CHUNK, NP), lambda k: (jnp.minimum(k, NCHUNK - 1), 0)),
            pl.BlockSpec((NP, NC), lambda k: (0, 0)),
            pl.BlockSpec((1, NC), lambda k: (0, 0)),
            pl.BlockSpec((G3, NC), lambda k: (0, 0)),
            pl.BlockSpec((1, G3), lambda k: (0, 0)),
            pl.BlockSpec((G3, H), lambda k: (0, 0)),
            pl.BlockSpec((1, G3), lambda k: (0, 0)),
        ],
        out_specs=pl.BlockSpec((B, H), lambda k: (0, 0)),
        out_shape=jax.ShapeDtypeStruct((B, H), f32),
        scratch_shapes=[pltpu.VMEM((2, TCH, B, G3), f32),
                        pltpu.VMEM((B, H), f32)],
    )(x_pad, g1, brow, gru_Wih, bih, whh_bf, bhh)

    rep = jnp.repeat(h_end, T, axis=1).reshape(B, T, H)
    rep_tb = rep.transpose(1, 0, 2).reshape(BT, H)

    out = pl.pallas_call(
        _dec_kernel,
        in_specs=[
            pl.BlockSpec((BT, H), lambda: (0, 0)),
            pl.BlockSpec((H, G3D), lambda: (0, 0)),
            pl.BlockSpec((1, G3D), lambda: (0, 0)),
            pl.BlockSpec((DP, G3D), lambda: (0, 0)),
            pl.BlockSpec((1, G3D), lambda: (0, 0)),
            pl.BlockSpec((DP, OUTP), lambda: (0, 0)),
            pl.BlockSpec((1, OUTP), lambda: (0, 0)),
        ],
        out_specs=pl.BlockSpec((BT, OUTP), lambda: (0, 0)),
        out_shape=jax.ShapeDtypeStruct((BT, OUTP), f32),
        scratch_shapes=[pltpu.VMEM((T, B, G3D), f32), pltpu.VMEM((T, B, DP), f32)],
    )(rep_tb, dwihT, dbih, dwhhT, dbhh, fcwT, fcb)

    return out[:, :OUT].reshape(T, B, OUT).transpose(1, 0, 2)


# speedup vs baseline: 1.1288x; 1.1288x over previous
import functools

import jax
import jax.numpy as jnp
from jax.experimental import pallas as pl
from jax.experimental.pallas import tpu as pltpu

N = 50; T = 100; B = 16; C = 32; H = 512; DH = 150; OUT = 50
NP = 64
NC = N * C
G3 = 3 * H
DP = 256
G3D = 3 * DP
OUTP = 128
BT = B * T
CHUNK = 64
NCHUNK = BT // CHUNK
TCH = CHUNK // B


def _prep_kernel(emb_ref, ew_ref, ev_ref, g1_ref):
    emb = emb_ref[...]
    sq = jnp.sum(emb * emb, axis=1, keepdims=True)
    norm = jnp.maximum(jnp.sqrt(sq), 1e-8)
    wn = emb / norm
    a = jax.lax.dot_general(wn, wn, (((1,), (1,)), ((), ())),
                            preferred_element_type=jnp.float32)
    ii = jax.lax.broadcasted_iota(jnp.int32, (NP, NP), 0)
    jj = jax.lax.broadcasted_iota(jnp.int32, (NP, NP), 1)
    a = jnp.where(ii == jj, 0.0, a)
    a = jnp.maximum(a, 0.0)
    deg = jnp.sum(a, axis=0, keepdims=True)
    dis = jnp.where(deg > 0, jax.lax.rsqrt(deg), 0.0)
    m = a * dis * jnp.transpose(dis)
    g1_ref[...] = jnp.dot(m, ew_ref[...],
                          preferred_element_type=jnp.float32) + ev_ref[...]


NSL = G3 // TCH


def _gemmenc_kernel(x_ref, g1_ref, brow_ref, wih_ref, bih_ref, whh_ref,
                    bhh_ref, hend_ref, gibuf_ref, h_ref):
    k = pl.program_id(0)

    @pl.when(k == 0)
    def _():
        h_ref[...] = jnp.zeros((B, H), jnp.float32)

    y = jnp.dot(x_ref[...], g1_ref[...], preferred_element_type=jnp.float32)
    y = y + brow_ref[...]
    g = (0.5 * y * (1.0 + jax.lax.erf(y * 0.7071067811865476))
         ).astype(jnp.bfloat16)

    def step(i, h):
        part = jnp.dot(g, wih_ref[:, pl.ds(i * NSL, NSL)],
                       preferred_element_type=jnp.float32)
        gibuf_ref[k % 2, :, :, pl.ds(i * NSL, NSL)] = (
            part + bih_ref[:, pl.ds(i * NSL, NSL)]).reshape(TCH, B, NSL)
        gi = gibuf_ref[(k - 1) % 2, i]
        gh = jnp.dot(h.astype(jnp.bfloat16), whh_ref[...],
                     preferred_element_type=jnp.float32) + bhh_ref[...]
        r = jax.nn.sigmoid(gi[:, :H] + gh[:, :H])
        z = jax.nn.sigmoid(gi[:, H:2 * H] + gh[:, H:2 * H])
        n = jnp.tanh(gi[:, 2 * H:] + r * gh[:, 2 * H:])
        return (1.0 - z) * n + z * h

    h = jax.lax.fori_loop(0, TCH, step, h_ref[...])
    h_ref[...] = jnp.where(k == 0, jnp.zeros((B, H), jnp.float32), h)

    @pl.when(k == NCHUNK)
    def _():
        hend_ref[...] = h_ref[...]


def _dec_kernel(rep_ref, dwih_ref, dbih_ref, dwhh_ref, dbhh_ref, fcw_ref,
                fcb_ref, out_ref, gid_ref, hs_ref):
    gid_ref[...] = (jnp.dot(rep_ref[...], dwih_ref[...],
                            preferred_element_type=jnp.float32)
                    + dbih_ref[...]).reshape(T, B, G3D)

    def dstep(t, hd):
        gi = gid_ref[t]
        ghd = jnp.dot(hd.astype(jnp.bfloat16), dwhh_ref[...],
                      preferred_element_type=jnp.float32) + dbhh_ref[...]
        r = jax.nn.sigmoid(gi[:, :DP] + ghd[:, :DP])
        z = jax.nn.sigmoid(gi[:, DP:2 * DP] + ghd[:, DP:2 * DP])
        n = jnp.tanh(gi[:, 2 * DP:] + r * ghd[:, 2 * DP:])
        hd = (1.0 - z) * n + z * hd
        hs_ref[t] = hd
        return hd

    jax.lax.fori_loop(0, T, dstep, jnp.zeros((B, DP), jnp.float32))
    out_ref[...] = jnp.dot(hs_ref[...].reshape(BT, DP), fcw_ref[...],
                           preferred_element_type=jnp.float32) + fcb_ref[...]


def _pad2(x, r, c):
    return jnp.pad(x, ((0, r - x.shape[0]), (0, c - x.shape[1])))


@jax.jit
def kernel(window, emb_W, arma_w, arma_v, arma_b, gru_Wih, gru_Whh, gru_bih,
           gru_bhh, dec_Wih, dec_Whh, dec_bih, dec_bhh, fc_W, fc_b):
    f32 = jnp.float32
    xtb = jnp.transpose(window, (1, 0, 2)).reshape(BT, N)
    x_pad = _pad2(xtb, BT, NP)
    emb_pad = _pad2(emb_W, NP, 128)
    eye = jnp.eye(N, dtype=f32)
    ew = _pad2((eye[:, :, None] * arma_w[0][None, None, :]).reshape(N, NC), NP, NC)
    ev = _pad2((eye[:, :, None] * arma_v[0][None, None, :]).reshape(N, NC), NP, NC)
    brow = jnp.tile(arma_b, N)[None, :]
    wihT = gru_Wih.T.astype(jnp.bfloat16)
    bih = gru_bih[None, :]
    whhT = gru_Whh.T.astype(jnp.bfloat16)
    bhh = gru_bhh[None, :]
    dwihT = jnp.concatenate(
        [_pad2(dec_Wih[g * DH:(g + 1) * DH, :].T, H, DP) for g in range(3)],
        axis=1)
    dbih = jnp.concatenate(
        [jnp.pad(dec_bih[g * DH:(g + 1) * DH], (0, DP - DH)) for g in range(3)]
    )[None, :]
    dwhhT = jnp.concatenate(
        [_pad2(dec_Whh[g * DH:(g + 1) * DH, :].T, DP, DP) for g in range(3)],
        axis=1).astype(jnp.bfloat16)
    dbhh = jnp.concatenate(
        [jnp.pad(dec_bhh[g * DH:(g + 1) * DH], (0, DP - DH)) for g in range(3)]
    )[None, :]
    fcwT = _pad2(fc_W.T, DP, OUTP)
    fcb = jnp.pad(fc_b, (0, OUTP - OUT))[None, :]

    g1 = pl.pallas_call(
        _prep_kernel,
        out_shape=jax.ShapeDtypeStruct((NP, NC), f32),
    )(emb_pad, ew, ev)

    h_end = pl.pallas_call(
        _gemmenc_kernel,
        grid=(NCHUNK + 1,),
        in_specs=[
            pl.BlockSpec((CHUNK, NP), lambda k: (jnp.minimum(k, NCHUNK - 1), 0)),
            pl.BlockSpec((NP, NC), lambda k: (0, 0)),
            pl.BlockSpec((1, NC), lambda k: (0, 0)),
            pl.BlockSpec((NC, G3), lambda k: (0, 0)),
            pl.BlockSpec((1, G3), lambda k: (0, 0)),
            pl.BlockSpec((H, G3), lambda k: (0, 0)),
            pl.BlockSpec((1, G3), lambda k: (0, 0)),
        ],
        out_specs=pl.BlockSpec((B, H), lambda k: (0, 0)),
        out_shape=jax.ShapeDtypeStruct((B, H), f32),
        scratch_shapes=[pltpu.VMEM((2, TCH, B, G3), f32),
                        pltpu.VMEM((B, H), f32)],
    )(x_pad, g1, brow, wihT, bih, whhT, bhh)

    rep = jnp.repeat(h_end, T, axis=1).reshape(B, T, H)
    rep_tb = rep.transpose(1, 0, 2).reshape(BT, H)

    out = pl.pallas_call(
        _dec_kernel,
        in_specs=[
            pl.BlockSpec((BT, H), lambda: (0, 0)),
            pl.BlockSpec((H, G3D), lambda: (0, 0)),
            pl.BlockSpec((1, G3D), lambda: (0, 0)),
            pl.BlockSpec((DP, G3D), lambda: (0, 0)),
            pl.BlockSpec((1, G3D), lambda: (0, 0)),
            pl.BlockSpec((DP, OUTP), lambda: (0, 0)),
            pl.BlockSpec((1, OUTP), lambda: (0, 0)),
        ],
        out_specs=pl.BlockSpec((BT, OUTP), lambda: (0, 0)),
        out_shape=jax.ShapeDtypeStruct((BT, OUTP), f32),
        scratch_shapes=[pltpu.VMEM((T, B, G3D), f32), pltpu.VMEM((T, B, DP), f32)],
    )(rep_tb, dwihT, dbih, dwhhT, dbhh, fcwT, fcb)

    return out[:, :OUT].reshape(T, B, OUT).transpose(1, 0, 2)


# speedup vs baseline: 1.3521x; 1.1978x over previous
import functools

import jax
import jax.numpy as jnp
from jax.experimental import pallas as pl
from jax.experimental.pallas import tpu as pltpu

N = 50; T = 100; B = 16; C = 32; H = 512; DH = 150; OUT = 50
NP = 64
NC = N * C
G3 = 3 * H
DP = 256
G3D = 3 * DP
OUTP = 128
BT = B * T
CHUNK = 160
NCHUNK = BT // CHUNK
TCH = CHUNK // B


def _prep_kernel(emb_ref, ew_ref, ev_ref, g1_ref):
    emb = emb_ref[...]
    sq = jnp.sum(emb * emb, axis=1, keepdims=True)
    norm = jnp.maximum(jnp.sqrt(sq), 1e-8)
    wn = emb / norm
    a = jax.lax.dot_general(wn, wn, (((1,), (1,)), ((), ())),
                            preferred_element_type=jnp.float32)
    ii = jax.lax.broadcasted_iota(jnp.int32, (NP, NP), 0)
    jj = jax.lax.broadcasted_iota(jnp.int32, (NP, NP), 1)
    a = jnp.where(ii == jj, 0.0, a)
    a = jnp.maximum(a, 0.0)
    deg = jnp.sum(a, axis=0, keepdims=True)
    dis = jnp.where(deg > 0, jax.lax.rsqrt(deg), 0.0)
    m = a * dis * jnp.transpose(dis)
    g1_ref[...] = jnp.dot(m, ew_ref[...],
                          preferred_element_type=jnp.float32) + ev_ref[...]


def _gemmenc_kernel(x_ref, g1_ref, brow_ref, wih_ref, bih_ref, whh_ref,
                    bhh_ref, hend_ref, gibuf_ref, h_ref):
    k = pl.program_id(0)

    @pl.when(k == 0)
    def _():
        h_ref[...] = jnp.zeros((B, H), jnp.float32)

    y = jnp.dot(x_ref[...], g1_ref[...], preferred_element_type=jnp.float32)
    y = y + brow_ref[...]
    g = (0.5 * y * (1.0 + jax.lax.erf(y * 0.7071067811865476))
         ).astype(jnp.bfloat16)
    gibuf_ref[k % 2] = (jnp.dot(g, wih_ref[...],
                                preferred_element_type=jnp.float32)
                        + bih_ref[...]).reshape(TCH, B, G3)

    h = h_ref[...]
    for i in range(TCH):
        gi = gibuf_ref[(k - 1) % 2, i]
        gh = jnp.dot(h.astype(jnp.bfloat16), whh_ref[...],
                     preferred_element_type=jnp.float32) + bhh_ref[...]
        r = jax.nn.sigmoid(gi[:, :H] + gh[:, :H])
        z = jax.nn.sigmoid(gi[:, H:2 * H] + gh[:, H:2 * H])
        n = jnp.tanh(gi[:, 2 * H:] + r * gh[:, 2 * H:])
        h = (1.0 - z) * n + z * h
    h_ref[...] = jnp.where(k == 0, jnp.zeros((B, H), jnp.float32), h)

    @pl.when(k == NCHUNK)
    def _():
        hend_ref[...] = h_ref[...]


def _dec_kernel(rep_ref, dwih_ref, dbih_ref, dwhh_ref, dbhh_ref, fcw_ref,
                fcb_ref, out_ref, gid_ref, hs_ref):
    gid_ref[...] = (jnp.dot(rep_ref[...], dwih_ref[...],
                            preferred_element_type=jnp.float32)
                    + dbih_ref[...]).reshape(T, B, G3D)

    def dstep(t, hd):
        gi = gid_ref[t]
        ghd = jnp.dot(hd.astype(jnp.bfloat16), dwhh_ref[...],
                      preferred_element_type=jnp.float32) + dbhh_ref[...]
        r = jax.nn.sigmoid(gi[:, :DP] + ghd[:, :DP])
        z = jax.nn.sigmoid(gi[:, DP:2 * DP] + ghd[:, DP:2 * DP])
        n = jnp.tanh(gi[:, 2 * DP:] + r * ghd[:, 2 * DP:])
        hd = (1.0 - z) * n + z * hd
        hs_ref[t] = hd
        return hd

    jax.lax.fori_loop(0, T, dstep, jnp.zeros((B, DP), jnp.float32))
    out_ref[...] = jnp.dot(hs_ref[...].reshape(BT, DP), fcw_ref[...],
                           preferred_element_type=jnp.float32) + fcb_ref[...]


def _pad2(x, r, c):
    return jnp.pad(x, ((0, r - x.shape[0]), (0, c - x.shape[1])))


@jax.jit
def kernel(window, emb_W, arma_w, arma_v, arma_b, gru_Wih, gru_Whh, gru_bih,
           gru_bhh, dec_Wih, dec_Whh, dec_bih, dec_bhh, fc_W, fc_b):
    f32 = jnp.float32
    xtb = jnp.transpose(window, (1, 0, 2)).reshape(BT, N)
    x_pad = _pad2(xtb, BT, NP)
    emb_pad = _pad2(emb_W, NP, 128)
    eye = jnp.eye(N, dtype=f32)
    ew = _pad2((eye[:, :, None] * arma_w[0][None, None, :]).reshape(N, NC), NP, NC)
    ev = _pad2((eye[:, :, None] * arma_v[0][None, None, :]).reshape(N, NC), NP, NC)
    brow = jnp.tile(arma_b, N)[None, :]
    wihT = gru_Wih.T.astype(jnp.bfloat16)
    bih = gru_bih[None, :]
    whhT = gru_Whh.T.astype(jnp.bfloat16)
    bhh = gru_bhh[None, :]
    dwihT = jnp.concatenate(
        [_pad2(dec_Wih[g * DH:(g + 1) * DH, :].T, H, DP) for g in range(3)],
        axis=1)
    dbih = jnp.concatenate(
        [jnp.pad(dec_bih[g * DH:(g + 1) * DH], (0, DP - DH)) for g in range(3)]
    )[None, :]
    dwhhT = jnp.concatenate(
        [_pad2(dec_Whh[g * DH:(g + 1) * DH, :].T, DP, DP) for g in range(3)],
        axis=1).astype(jnp.bfloat16)
    dbhh = jnp.concatenate(
        [jnp.pad(dec_bhh[g * DH:(g + 1) * DH], (0, DP - DH)) for g in range(3)]
    )[None, :]
    fcwT = _pad2(fc_W.T, DP, OUTP)
    fcb = jnp.pad(fc_b, (0, OUTP - OUT))[None, :]

    g1 = pl.pallas_call(
        _prep_kernel,
        out_shape=jax.ShapeDtypeStruct((NP, NC), f32),
    )(emb_pad, ew, ev)

    h_end = pl.pallas_call(
        _gemmenc_kernel,
        grid=(NCHUNK + 1,),
        in_specs=[
            pl.BlockSpec((CHUNK, NP), lambda k: (jnp.minimum(k, NCHUNK - 1), 0)),
            pl.BlockSpec((NP, NC), lambda k: (0, 0)),
            pl.BlockSpec((1, NC), lambda k: (0, 0)),
            pl.BlockSpec((NC, G3), lambda k: (0, 0)),
            pl.BlockSpec((1, G3), lambda k: (0, 0)),
            pl.BlockSpec((H, G3), lambda k: (0, 0)),
            pl.BlockSpec((1, G3), lambda k: (0, 0)),
        ],
        out_specs=pl.BlockSpec((B, H), lambda k: (0, 0)),
        out_shape=jax.ShapeDtypeStruct((B, H), f32),
        scratch_shapes=[pltpu.VMEM((2, TCH, B, G3), f32),
                        pltpu.VMEM((B, H), f32)],
    )(x_pad, g1, brow, wihT, bih, whhT, bhh)

    rep = jnp.repeat(h_end, T, axis=1).reshape(B, T, H)
    rep_tb = rep.transpose(1, 0, 2).reshape(BT, H)

    out = pl.pallas_call(
        _dec_kernel,
        in_specs=[
            pl.BlockSpec((BT, H), lambda: (0, 0)),
            pl.BlockSpec((H, G3D), lambda: (0, 0)),
            pl.BlockSpec((1, G3D), lambda: (0, 0)),
            pl.BlockSpec((DP, G3D), lambda: (0, 0)),
            pl.BlockSpec((1, G3D), lambda: (0, 0)),
            pl.BlockSpec((DP, OUTP), lambda: (0, 0)),
            pl.BlockSpec((1, OUTP), lambda: (0, 0)),
        ],
        out_specs=pl.BlockSpec((BT, OUTP), lambda: (0, 0)),
        out_shape=jax.ShapeDtypeStruct((BT, OUTP), f32),
        scratch_shapes=[pltpu.VMEM((T, B, G3D), f32), pltpu.VMEM((T, B, DP), f32)],
    )(rep_tb, dwihT, dbih, dwhhT, dbhh, fcwT, fcb)

    return out[:, :OUT].reshape(T, B, OUT).transpose(1, 0, 2)


# speedup vs baseline: 1.3531x; 1.0007x over previous
import functools

import jax
import jax.numpy as jnp
from jax.experimental import pallas as pl
from jax.experimental.pallas import tpu as pltpu

N = 50; T = 100; B = 16; C = 32; H = 512; DH = 150; OUT = 50
NP = 64
NC = N * C
G3 = 3 * H
DP = 256
G3D = 3 * DP
OUTP = 128
BT = B * T
CHUNK = 160
NCHUNK = BT // CHUNK
TCH = CHUNK // B


def _prep_kernel(emb_ref, ew_ref, ev_ref, g1_ref):
    emb = emb_ref[...]
    sq = jnp.sum(emb * emb, axis=1, keepdims=True)
    norm = jnp.maximum(jnp.sqrt(sq), 1e-8)
    wn = emb / norm
    a = jax.lax.dot_general(wn, wn, (((1,), (1,)), ((), ())),
                            preferred_element_type=jnp.float32)
    ii = jax.lax.broadcasted_iota(jnp.int32, (NP, NP), 0)
    jj = jax.lax.broadcasted_iota(jnp.int32, (NP, NP), 1)
    a = jnp.where(ii == jj, 0.0, a)
    a = jnp.maximum(a, 0.0)
    deg = jnp.sum(a, axis=0, keepdims=True)
    dis = jnp.where(deg > 0, jax.lax.rsqrt(deg), 0.0)
    m = a * dis * jnp.transpose(dis)
    g1_ref[...] = jnp.dot(m, ew_ref[...],
                          preferred_element_type=jnp.float32) + ev_ref[...]


def _gemmenc_kernel(x_ref, g1_ref, brow_ref, wih_ref, bih_ref, whh_ref,
                    bhh_ref, hend_ref, gibuf_ref, h_ref):
    k = pl.program_id(0)

    @pl.when(k == 0)
    def _():
        h_ref[...] = jnp.zeros((B, H), jnp.float32)

    y = jnp.dot(x_ref[...], g1_ref[...], preferred_element_type=jnp.float32)
    y = y + brow_ref[...]
    g = (0.5 * y * (1.0 + jax.lax.erf(y * 0.7071067811865476))
         ).astype(jnp.bfloat16)
    gibuf_ref[k % 2] = (jnp.dot(g, wih_ref[...],
                                preferred_element_type=jnp.float32)
                        + bih_ref[...]).reshape(TCH, B, G3)

    h = h_ref[...]
    for i in range(TCH):
        gi = gibuf_ref[(k - 1) % 2, i]
        hb = h.astype(jnp.bfloat16)
        gh_r = jnp.dot(hb, whh_ref[:, :H], preferred_element_type=jnp.float32)
        gh_n = jnp.dot(hb, whh_ref[:, 2 * H:],
                       preferred_element_type=jnp.float32)
        gh_z = jnp.dot(hb, whh_ref[:, H:2 * H],
                       preferred_element_type=jnp.float32)
        r = jax.nn.sigmoid(gi[:, :H] + gh_r + bhh_ref[:, :H])
        z = jax.nn.sigmoid(gi[:, H:2 * H] + gh_z + bhh_ref[:, H:2 * H])
        n = jnp.tanh(gi[:, 2 * H:] + r * (gh_n + bhh_ref[:, 2 * H:]))
        h = (1.0 - z) * n + z * h
    h_ref[...] = jnp.where(k == 0, jnp.zeros((B, H), jnp.float32), h)

    @pl.when(k == NCHUNK)
    def _():
        hend_ref[...] = h_ref[...]


def _dec_kernel(rep_ref, dwih_ref, dbih_ref, dwhh_ref, dbhh_ref, fcw_ref,
                fcb_ref, out_ref, gid_ref, hs_ref):
    gid_ref[...] = (jnp.dot(rep_ref[...], dwih_ref[...],
                            preferred_element_type=jnp.float32)
                    + dbih_ref[...]).reshape(T, B, G3D)

    def dstep(t, hd):
        gi = gid_ref[t]
        hb = hd.astype(jnp.bfloat16)
        gh_r = jnp.dot(hb, dwhh_ref[:, :DP], preferred_element_type=jnp.float32)
        gh_n = jnp.dot(hb, dwhh_ref[:, 2 * DP:],
                       preferred_element_type=jnp.float32)
        gh_z = jnp.dot(hb, dwhh_ref[:, DP:2 * DP],
                       preferred_element_type=jnp.float32)
        r = jax.nn.sigmoid(gi[:, :DP] + gh_r + dbhh_ref[:, :DP])
        z = jax.nn.sigmoid(gi[:, DP:2 * DP] + gh_z + dbhh_ref[:, DP:2 * DP])
        n = jnp.tanh(gi[:, 2 * DP:] + r * (gh_n + dbhh_ref[:, 2 * DP:]))
        hd = (1.0 - z) * n + z * hd
        hs_ref[t] = hd
        return hd

    jax.lax.fori_loop(0, T, dstep, jnp.zeros((B, DP), jnp.float32))
    out_ref[...] = jnp.dot(hs_ref[...].reshape(BT, DP), fcw_ref[...],
                           preferred_element_type=jnp.float32) + fcb_ref[...]


def _pad2(x, r, c):
    return jnp.pad(x, ((0, r - x.shape[0]), (0, c - x.shape[1])))


@jax.jit
def kernel(window, emb_W, arma_w, arma_v, arma_b, gru_Wih, gru_Whh, gru_bih,
           gru_bhh, dec_Wih, dec_Whh, dec_bih, dec_bhh, fc_W, fc_b):
    f32 = jnp.float32
    xtb = jnp.transpose(window, (1, 0, 2)).reshape(BT, N)
    x_pad = _pad2(xtb, BT, NP)
    emb_pad = _pad2(emb_W, NP, 128)
    eye = jnp.eye(N, dtype=f32)
    ew = _pad2((eye[:, :, None] * arma_w[0][None, None, :]).reshape(N, NC), NP, NC)
    ev = _pad2((eye[:, :, None] * arma_v[0][None, None, :]).reshape(N, NC), NP, NC)
    brow = jnp.tile(arma_b, N)[None, :]
    wihT = gru_Wih.T.astype(jnp.bfloat16)
    bih = gru_bih[None, :]
    whhT = gru_Whh.T.astype(jnp.bfloat16)
    bhh = gru_bhh[None, :]
    dwihT = jnp.concatenate(
        [_pad2(dec_Wih[g * DH:(g + 1) * DH, :].T, H, DP) for g in range(3)],
        axis=1)
    dbih = jnp.concatenate(
        [jnp.pad(dec_bih[g * DH:(g + 1) * DH], (0, DP - DH)) for g in range(3)]
    )[None, :]
    dwhhT = jnp.concatenate(
        [_pad2(dec_Whh[g * DH:(g + 1) * DH, :].T, DP, DP) for g in range(3)],
        axis=1).astype(jnp.bfloat16)
    dbhh = jnp.concatenate(
        [jnp.pad(dec_bhh[g * DH:(g + 1) * DH], (0, DP - DH)) for g in range(3)]
    )[None, :]
    fcwT = _pad2(fc_W.T, DP, OUTP)
    fcb = jnp.pad(fc_b, (0, OUTP - OUT))[None, :]

    g1 = pl.pallas_call(
        _prep_kernel,
        out_shape=jax.ShapeDtypeStruct((NP, NC), f32),
    )(emb_pad, ew, ev)

    h_end = pl.pallas_call(
        _gemmenc_kernel,
        grid=(NCHUNK + 1,),
        in_specs=[
            pl.BlockSpec((CHUNK, NP), lambda k: (jnp.minimum(k, NCHUNK - 1), 0)),
            pl.BlockSpec((NP, NC), lambda k: (0, 0)),
            pl.BlockSpec((1, NC), lambda k: (0, 0)),
            pl.BlockSpec((NC, G3), lambda k: (0, 0)),
            pl.BlockSpec((1, G3), lambda k: (0, 0)),
            pl.BlockSpec((H, G3), lambda k: (0, 0)),
            pl.BlockSpec((1, G3), lambda k: (0, 0)),
        ],
        out_specs=pl.BlockSpec((B, H), lambda k: (0, 0)),
        out_shape=jax.ShapeDtypeStruct((B, H), f32),
        scratch_shapes=[pltpu.VMEM((2, TCH, B, G3), f32),
                        pltpu.VMEM((B, H), f32)],
    )(x_pad, g1, brow, wihT, bih, whhT, bhh)

    rep = jnp.repeat(h_end, T, axis=1).reshape(B, T, H)
    rep_tb = rep.transpose(1, 0, 2).reshape(BT, H)

    out = pl.pallas_call(
        _dec_kernel,
        in_specs=[
            pl.BlockSpec((BT, H), lambda: (0, 0)),
            pl.BlockSpec((H, G3D), lambda: (0, 0)),
            pl.BlockSpec((1, G3D), lambda: (0, 0)),
            pl.BlockSpec((DP, G3D), lambda: (0, 0)),
            pl.BlockSpec((1, G3D), lambda: (0, 0)),
            pl.BlockSpec((DP, OUTP), lambda: (0, 0)),
            pl.BlockSpec((1, OUTP), lambda: (0, 0)),
        ],
        out_specs=pl.BlockSpec((BT, OUTP), lambda: (0, 0)),
        out_shape=jax.ShapeDtypeStruct((BT, OUTP), f32),
        scratch_shapes=[pltpu.VMEM((T, B, G3D), f32), pltpu.VMEM((T, B, DP), f32)],
    )(rep_tb, dwihT, dbih, dwhhT, dbhh, fcwT, fcb)

    return out[:, :OUT].reshape(T, B, OUT).transpose(1, 0, 2)


# speedup vs baseline: 1.3673x; 1.0105x over previous
import functools

import jax
import jax.numpy as jnp
import numpy as np
from jax.experimental import pallas as pl
from jax.experimental.pallas import tpu as pltpu

N = 50; T = 100; B = 16; C = 32; H = 512; DH = 150; OUT = 50
NP = 64
NC = N * C
G3 = 3 * H
DP = 256
G3D = 3 * DP
OUTP = 128
BT = B * T
CHUNK = 160
NCHUNK = BT // CHUNK
TCH = CHUNK // B

WW = 8
_K0 = (np.arange(T) * H) // T
_KM = _K0[:, None] + np.arange(WW)[None, :]
_LO = np.clip(_KM * T - (np.arange(T) * H)[:, None], 0, H)
_HI = np.clip(_KM * T + T - (np.arange(T) * H)[:, None], 0, H)
_WIDX = np.clip(_KM, 0, H - 1)


def _prep_kernel(emb_ref, ew_ref, ev_ref, g1_ref):
    emb = emb_ref[...]
    sq = jnp.sum(emb * emb, axis=1, keepdims=True)
    norm = jnp.maximum(jnp.sqrt(sq), 1e-8)
    wn = emb / norm
    a = jax.lax.dot_general(wn, wn, (((1,), (1,)), ((), ())),
                            preferred_element_type=jnp.float32)
    ii = jax.lax.broadcasted_iota(jnp.int32, (NP, NP), 0)
    jj = jax.lax.broadcasted_iota(jnp.int32, (NP, NP), 1)
    a = jnp.where(ii == jj, 0.0, a)
    a = jnp.maximum(a, 0.0)
    deg = jnp.sum(a, axis=0, keepdims=True)
    dis = jnp.where(deg > 0, jax.lax.rsqrt(deg), 0.0)
    m = a * dis * jnp.transpose(dis)
    g1_ref[...] = jnp.dot(m, ew_ref[...],
                          preferred_element_type=jnp.float32) + ev_ref[...]


def _gemmenc_kernel(x_ref, g1_ref, brow_ref, wih_ref, bih_ref, whh_ref,
                    bhh_ref, hend_ref, gibuf_ref, h_ref):
    k = pl.program_id(0)

    @pl.when(k == 0)
    def _():
        h_ref[...] = jnp.zeros((B, H), jnp.float32)

    y = jnp.dot(x_ref[...], g1_ref[...], preferred_element_type=jnp.float32)
    y = y + brow_ref[...]
    g = (0.5 * y * (1.0 + jax.lax.erf(y * 0.7071067811865476))
         ).astype(jnp.bfloat16)
    gibuf_ref[k % 2] = (jnp.dot(g, wih_ref[...],
                                preferred_element_type=jnp.float32)
                        + bih_ref[...]).reshape(TCH, B, G3)

    h = h_ref[...]
    for i in range(TCH):
        gi = gibuf_ref[(k - 1) % 2, i]
        hb = h.astype(jnp.bfloat16)
        gh_r = jnp.dot(hb, whh_ref[:, :H], preferred_element_type=jnp.float32)
        gh_n = jnp.dot(hb, whh_ref[:, 2 * H:],
                       preferred_element_type=jnp.float32)
        gh_z = jnp.dot(hb, whh_ref[:, H:2 * H],
                       preferred_element_type=jnp.float32)
        r = jax.nn.sigmoid(gi[:, :H] + gh_r + bhh_ref[:, :H])
        z = jax.nn.sigmoid(gi[:, H:2 * H] + gh_z + bhh_ref[:, H:2 * H])
        n = jnp.tanh(gi[:, 2 * H:] + r * (gh_n + bhh_ref[:, 2 * H:]))
        h = (1.0 - z) * n + z * h
    h_ref[...] = jnp.where(k == 0, jnp.zeros((B, H), jnp.float32), h)

    @pl.when(k == NCHUNK)
    def _():
        hend_ref[...] = h_ref[...]


def _dec_kernel(hwin_ref, uwin_ref, dbih_ref, dwhh_ref, dbhh_ref, fcw_ref,
                fcb_ref, out_ref, hs_ref):
    def dstep(t, hd):
        gi = jnp.dot(hwin_ref[t], uwin_ref[t],
                     preferred_element_type=jnp.float32) + dbih_ref[...]
        hb = hd.astype(jnp.bfloat16)
        gh_r = jnp.dot(hb, dwhh_ref[:, :DP], preferred_element_type=jnp.float32)
        gh_n = jnp.dot(hb, dwhh_ref[:, 2 * DP:],
                       preferred_element_type=jnp.float32)
        gh_z = jnp.dot(hb, dwhh_ref[:, DP:2 * DP],
                       preferred_element_type=jnp.float32)
        r = jax.nn.sigmoid(gi[:, :DP] + gh_r + dbhh_ref[:, :DP])
        z = jax.nn.sigmoid(gi[:, DP:2 * DP] + gh_z + dbhh_ref[:, DP:2 * DP])
        n = jnp.tanh(gi[:, 2 * DP:] + r * (gh_n + dbhh_ref[:, 2 * DP:]))
        hd = (1.0 - z) * n + z * hd
        hs_ref[t] = hd
        return hd

    jax.lax.fori_loop(0, T, dstep, jnp.zeros((B, DP), jnp.float32))
    out_ref[...] = jnp.dot(hs_ref[...].reshape(BT, DP), fcw_ref[...],
                           preferred_element_type=jnp.float32) + fcb_ref[...]


def _pad2(x, r, c):
    return jnp.pad(x, ((0, r - x.shape[0]), (0, c - x.shape[1])))


@jax.jit
def kernel(window, emb_W, arma_w, arma_v, arma_b, gru_Wih, gru_Whh, gru_bih,
           gru_bhh, dec_Wih, dec_Whh, dec_bih, dec_bhh, fc_W, fc_b):
    f32 = jnp.float32
    xtb = jnp.transpose(window, (1, 0, 2)).reshape(BT, N)
    x_pad = _pad2(xtb, BT, NP)
    emb_pad = _pad2(emb_W, NP, 128)
    eye = jnp.eye(N, dtype=f32)
    ew = _pad2((eye[:, :, None] * arma_w[0][None, None, :]).reshape(N, NC), NP, NC)
    ev = _pad2((eye[:, :, None] * arma_v[0][None, None, :]).reshape(N, NC), NP, NC)
    brow = jnp.tile(arma_b, N)[None, :]
    wihT = gru_Wih.T.astype(jnp.bfloat16)
    bih = gru_bih[None, :]
    whhT = gru_Whh.T.astype(jnp.bfloat16)
    bhh = gru_bhh[None, :]
    dwihT = jnp.concatenate(
        [_pad2(dec_Wih[g * DH:(g + 1) * DH, :].T, H, DP) for g in range(3)],
        axis=1)
    dbih = jnp.concatenate(
        [jnp.pad(dec_bih[g * DH:(g + 1) * DH], (0, DP - DH)) for g in range(3)]
    )[None, :]
    dwhhT = jnp.concatenate(
        [_pad2(dec_Whh[g * DH:(g + 1) * DH, :].T, DP, DP) for g in range(3)],
        axis=1).astype(jnp.bfloat16)
    dbhh = jnp.concatenate(
        [jnp.pad(dec_bhh[g * DH:(g + 1) * DH], (0, DP - DH)) for g in range(3)]
    )[None, :]
    fcwT = _pad2(fc_W.T, DP, OUTP)
    fcb = jnp.pad(fc_b, (0, OUTP - OUT))[None, :]

    g1 = pl.pallas_call(
        _prep_kernel,
        out_shape=jax.ShapeDtypeStruct((NP, NC), f32),
    )(emb_pad, ew, ev)

    h_end = pl.pallas_call(
        _gemmenc_kernel,
        grid=(NCHUNK + 1,),
        in_specs=[
            pl.BlockSpec((CHUNK, NP), lambda k: (jnp.minimum(k, NCHUNK - 1), 0)),
            pl.BlockSpec((NP, NC), lambda k: (0, 0)),
            pl.BlockSpec((1, NC), lambda k: (0, 0)),
            pl.BlockSpec((NC, G3), lambda k: (0, 0)),
            pl.BlockSpec((1, G3), lambda k: (0, 0)),
            pl.BlockSpec((H, G3), lambda k: (0, 0)),
            pl.BlockSpec((1, G3), lambda k: (0, 0)),
        ],
        out_specs=pl.BlockSpec((B, H), lambda k: (0, 0)),
        out_shape=jax.ShapeDtypeStruct((B, H), f32),
        scratch_shapes=[pltpu.VMEM((2, TCH, B, G3), f32),
                        pltpu.VMEM((B, H), f32)],
    )(x_pad, g1, brow, wihT, bih, whhT, bhh)

    P = jnp.concatenate([jnp.zeros((1, G3D), f32), jnp.cumsum(dwihT, axis=0)])
    uwin = P[_HI] - P[_LO]
    hwin = jnp.transpose(h_end[:, _WIDX], (1, 0, 2))

    out = pl.pallas_call(
        _dec_kernel,
        in_specs=[
            pl.BlockSpec((T, B, WW), lambda: (0, 0, 0)),
            pl.BlockSpec((T, WW, G3D), lambda: (0, 0, 0)),
            pl.BlockSpec((1, G3D), lambda: (0, 0)),
            pl.BlockSpec((DP, G3D), lambda: (0, 0)),
            pl.BlockSpec((1, G3D), lambda: (0, 0)),
            pl.BlockSpec((DP, OUTP), lambda: (0, 0)),
            pl.BlockSpec((1, OUTP), lambda: (0, 0)),
        ],
        out_specs=pl.BlockSpec((BT, OUTP), lambda: (0, 0)),
        out_shape=jax.ShapeDtypeStruct((BT, OUTP), f32),
        scratch_shapes=[pltpu.VMEM((T, B, DP), f32)],
    )(hwin, uwin, dbih, dwhhT, dbhh, fcwT, fcb)

    return out[:, :OUT].reshape(T, B, OUT).transpose(1, 0, 2)
